# Initial kernel scaffold; baseline (speedup 1.0000x reference)
import jax
import jax.numpy as jnp
from jax.experimental import pallas as pl


def kernel(x, codebooks):
    raise NotImplementedError("write your pallas kernel here")


# speedup vs baseline: 1.9703x; 1.9703x over previous
import functools

import jax
import jax.numpy as jnp
from jax.experimental import pallas as pl
from jax.experimental.pallas import tpu as pltpu

_Q = 3
_K = 256
_D = 32
_B = 64
_N = 1024
_T = _B * _N
_TBLK = 2048
_NBLK = _T // _TBLK
_COUNT = _T * _D


def _rvq_block(x_ref, cb_ref, quant_ref, idx_ref, loss_ref):
    b = pl.program_id(0)

    @pl.when(b == 0)
    def _init():
        loss_ref[...] = jnp.zeros_like(loss_ref)

    r = x_ref[...]
    quant = jnp.zeros_like(r)
    iota_k = jax.lax.broadcasted_iota(jnp.int32, (_TBLK, _K), 1)
    lane_iota = jax.lax.broadcasted_iota(jnp.int32, (1, 128), 1)
    loss_acc = loss_ref[...]
    for i in range(_Q):
        c = cb_ref[i]
        n_c = jnp.sum(c * c, axis=-1)
        s_r = jnp.sum(r * r, axis=-1, keepdims=True)
        s = jax.lax.dot_general(
            r, c, (((1,), (1,)), ((), ())),
            precision=jax.lax.Precision.DEFAULT,
            preferred_element_type=jnp.float32)
        d = s_r - 2.0 * s + n_c[None, :]
        m = jnp.min(d, axis=-1, keepdims=True)
        idx = jnp.min(jnp.where(d == m, iota_k, _K), axis=-1)
        onehot = (iota_k == idx[:, None]).astype(jnp.float32)
        q = jax.lax.dot_general(
            onehot, c, (((1,), (0,)), ((), ())),
            precision=jax.lax.Precision.HIGHEST,
            preferred_element_type=jnp.float32)
        q_st = r + (q - r)
        sumsq = jnp.sum((q - r) * (q - r)) * (1.0 / _COUNT)
        loss_acc = loss_acc + jnp.where(lane_iota == i, sumsq, 0.0)
        quant = quant + q_st
        r = r - q_st
        idx_ref[0, i, :] = idx
    quant_ref[...] = quant
    loss_ref[...] = loss_acc


@functools.partial(jax.jit, static_argnames=())
def kernel(x, codebooks):
    xf = x.reshape(_T, _D)
    grid = (_NBLK,)
    quant, idx, loss = pl.pallas_call(
        _rvq_block,
        grid=grid,
        in_specs=[
            pl.BlockSpec((_TBLK, _D), lambda b: (b, 0)),
            pl.BlockSpec((_Q, _K, _D), lambda b: (0, 0, 0)),
        ],
        out_specs=[
            pl.BlockSpec((_TBLK, _D), lambda b: (b, 0)),
            pl.BlockSpec((1, _Q, _TBLK), lambda b: (b, 0, 0)),
            pl.BlockSpec((1, 128), lambda b: (0, 0)),
        ],
        out_shape=[
            jax.ShapeDtypeStruct((_T, _D), jnp.float32),
            jax.ShapeDtypeStruct((_NBLK, _Q, _TBLK), jnp.int32),
            jax.ShapeDtypeStruct((1, 128), jnp.float32),
        ],
        compiler_params=pltpu.CompilerParams(
            dimension_semantics=("arbitrary",),
        ),
    )(xf, codebooks)
    quantized = quant.reshape(_B, _N, _D)
    indices = idx.transpose(0, 2, 1).reshape(_B, _N, _Q)
    commit_loss = loss[0, :_Q]
    return (quantized, indices, commit_loss)


# speedup vs baseline: 6.5149x; 3.3065x over previous
import functools

import jax
import jax.numpy as jnp
from jax.experimental import pallas as pl
from jax.experimental.pallas import tpu as pltpu

_Q = 3
_K = 256
_D = 32
_B = 64
_N = 1024
_T = _B * _N
_TBLK = 2048
_NBLK = _T // _TBLK
_COUNT = _T * _D


def _rvq_block(xt_ref, cb_ref, cbt_ref, quant_ref, idx_ref, loss_ref):
    b = pl.program_id(0)

    @pl.when(b == 0)
    def _init():
        loss_ref[...] = jnp.zeros_like(loss_ref)

    rt = xt_ref[...]
    quant = jnp.zeros_like(rt)
    iota_k = jax.lax.broadcasted_iota(jnp.int32, (_K, _TBLK), 0)
    lane_iota = jax.lax.broadcasted_iota(jnp.int32, (1, 128), 1)
    loss_acc = loss_ref[...]
    for i in range(_Q):
        c = cb_ref[i]
        ct = cbt_ref[i]
        n_c = jnp.sum(c * c, axis=-1).reshape(_K, 1)
        sq = rt * rt
        h = sq[0:16, :] + sq[16:32, :]
        h = h[0:8, :] + h[8:16, :]
        s_r = jnp.sum(h, axis=0, keepdims=True)
        s = jax.lax.dot_general(
            c * -2.0, rt, (((1,), (0,)), ((), ())),
            precision=jax.lax.Precision.DEFAULT,
            preferred_element_type=jnp.float32)
        d = (s_r + s) + n_c
        m = jnp.min(d, axis=0, keepdims=True)
        idx = jnp.min(jnp.where(d == m, iota_k, _K), axis=0)
        idxb = jnp.broadcast_to(idx[None, :], (_D, _TBLK))
        ilow = jnp.bitwise_and(idxb, 127)
        q0 = jnp.take_along_axis(ct[:, 0:128], ilow, axis=1)
        q1 = jnp.take_along_axis(ct[:, 128:256], ilow, axis=1)
        q = jnp.where(idxb < 128, q0, q1)
        q_st = rt + (q - rt)
        sumsq = jnp.sum((q - rt) * (q - rt)) * (1.0 / _COUNT)
        loss_acc = loss_acc + jnp.where(lane_iota == i, sumsq, 0.0)
        quant = quant + q_st
        rt = rt - q_st
        idx_ref[0, i, :] = idx
    quant_ref[...] = quant
    loss_ref[...] = loss_acc


@functools.partial(jax.jit, static_argnames=())
def kernel(x, codebooks):
    xt = x.reshape(_T, _D).T
    cbt = codebooks.transpose(0, 2, 1)
    grid = (_NBLK,)
    quant, idx, loss = pl.pallas_call(
        _rvq_block,
        grid=grid,
        in_specs=[
            pl.BlockSpec((_D, _TBLK), lambda b: (0, b)),
            pl.BlockSpec((_Q, _K, _D), lambda b: (0, 0, 0)),
            pl.BlockSpec((_Q, _D, _K), lambda b: (0, 0, 0)),
        ],
        out_specs=[
            pl.BlockSpec((_D, _TBLK), lambda b: (0, b)),
            pl.BlockSpec((1, _Q, _TBLK), lambda b: (b, 0, 0)),
            pl.BlockSpec((1, 128), lambda b: (0, 0)),
        ],
        out_shape=[
            jax.ShapeDtypeStruct((_D, _T), jnp.float32),
            jax.ShapeDtypeStruct((_NBLK, _Q, _TBLK), jnp.int32),
            jax.ShapeDtypeStruct((1, 128), jnp.float32),
        ],
        compiler_params=pltpu.CompilerParams(
            dimension_semantics=("arbitrary",),
        ),
    )(xt, codebooks, cbt)
    quantized = quant.T.reshape(_B, _N, _D)
    indices = idx.transpose(0, 2, 1).reshape(_B, _N, _Q)
    commit_loss = loss[0, :_Q]
    return (quantized, indices, commit_loss)
